# baseline (speedup 1.0000x reference)
import jax
import jax.numpy as jnp
from jax import lax
from jax.experimental import pallas as pl
from jax.experimental.pallas import tpu as pltpu

WSZ = 256
TOKEN_SELF_ATTN_VALUE = -50000.0
MININT = -2147483648
NOT_SELECTED = 1 << 20


def _excl_cumsum_cols(x, chunk=512):
    t = x.shape[0]
    r_i = lax.broadcasted_iota(jnp.int32, (chunk, chunk), 0)
    c_i = lax.broadcasted_iota(jnp.int32, (chunk, chunk), 1)
    lstrict = (c_i < r_i).astype(jnp.float32)
    carry = jnp.zeros((1, x.shape[1]), jnp.float32)
    outs = []
    for b0 in range(0, t, chunk):
        xb = lax.slice(x, (b0, 0), (b0 + chunk, x.shape[1]))
        outs.append(jnp.dot(lstrict, xb) + carry)
        carry = carry + jnp.sum(xb, axis=0, keepdims=True)
    return jnp.concatenate(outs, axis=0)


def _roll_left(x, s):
    n = x.shape[-1]
    s = s % n
    if s == 0:
        return x
    return jnp.concatenate(
        [lax.slice_in_dim(x, s, n, axis=1), lax.slice_in_dim(x, 0, s, axis=1)],
        axis=1)


def _route_kernel(qk_ref, means_t_ref, ranksel_ref, count_ref):
    t, d = qk_ref.shape[2], qk_ref.shape[3]
    wsz = WSZ
    nc = t // wsz

    qk = qk_ref[0, 0]
    means_t = means_t_ref[0]

    nrm = jnp.sqrt(jnp.sum(qk * qk, axis=1, keepdims=True))
    kn = qk / jnp.maximum(nrm, 1e-12)
    dists = jnp.dot(kn, means_t)

    u = lax.bitcast_convert_type(dists, jnp.int32)
    key = u ^ (lax.shift_right_arithmetic(u, 31) & jnp.int32(0x7FFFFFFF))

    tb = jnp.zeros((1, nc), jnp.int32)
    minint = jnp.int32(MININT)
    for bit in range(31, -1, -1):
        step = minint if bit == 31 else jnp.int32(1 << bit)
        try_b = tb + step
        t_signed = try_b ^ minint
        cnt = jnp.sum((key >= t_signed).astype(jnp.float32), axis=0,
                      keepdims=True)
        tb = jnp.where(cnt >= float(wsz), try_b, tb)
    thr = tb ^ minint

    gt = key > thr
    eq = key == thr
    n_gt = jnp.sum(gt.astype(jnp.float32), axis=0, keepdims=True)
    need = float(wsz) - n_gt
    eq_rank = _excl_cumsum_cols(eq.astype(jnp.float32))
    sel = gt | (eq & (eq_rank < need))
    sel_f = sel.astype(jnp.float32)
    rank = _excl_cumsum_cols(sel_f)

    ranksel_ref[0, 0] = jnp.where(sel, rank.astype(jnp.int32),
                                  jnp.int32(NOT_SELECTED))
    count_ref[0, 0] = jnp.sum(sel_f, axis=1, keepdims=True)


def _attn_kernel(ranksel_ref, qk_ref, v_ref, rel_t_ref, count_ref, out_ref,
                 acc_ref):
    t, d = qk_ref.shape[2], qk_ref.shape[3]
    wsz = WSZ
    nc = t // wsz
    scale = d ** -0.5
    c = pl.program_id(2)

    qk = qk_ref[0, 0]
    v = v_ref[0, 0]
    rel = rel_t_ref[0]
    rank_c = ranksel_ref[0, 0, 0]

    w_iota = lax.broadcasted_iota(jnp.int32, (t, wsz), 1)
    p_mat = jnp.where(rank_c == w_iota, 1.0, 0.0)

    qk_s = lax.dot_general(p_mat, qk, (((0,), (0,)), ((), ())))
    v_s = lax.dot_general(p_mat, v, (((0,), (0,)), ((), ())))

    knrm = jnp.sqrt(jnp.sum(qk_s * qk_s, axis=1, keepdims=True))
    kk = qk_s / jnp.maximum(knrm, 1e-12)
    dots = lax.dot_general(qk_s, kk, (((1,), (1,)), ((), ()))) * scale
    emb = lax.dot_general(qk_s, rel, (((1,), (1,)), ((), ()))) * scale

    rows = lax.broadcasted_iota(jnp.int32, (wsz, wsz), 0)
    cols = lax.broadcasted_iota(jnp.int32, (wsz, wsz), 1)
    x = emb
    for kbit in range(8):
        rolled = _roll_left(x, 1 << kbit)
        condb = ((((wsz - 1) - rows) >> kbit) & 1) == 1
        x = jnp.where(condb, rolled, x)
    dots = dots + jnp.where(cols <= rows, x, 0.0)
    dots = jnp.where(rows == cols, TOKEN_SELF_ATTN_VALUE, dots)

    m = jnp.max(dots, axis=1, keepdims=True)
    p = jnp.exp(dots - m)
    sm = p / jnp.sum(p, axis=1, keepdims=True)
    bo = jnp.dot(sm, v_s)
    contrib = jnp.dot(p_mat, bo)

    @pl.when(c == 0)
    def _():
        acc_ref[...] = contrib

    @pl.when(c > 0)
    def _():
        acc_ref[...] = acc_ref[...] + contrib

    @pl.when(c == nc - 1)
    def _():
        out_ref[0, 0] = acc_ref[...] / (count_ref[0, 0] + 1e-05)


def _run(qk, v, means_t, rel_t, interpret=False):
    b, h, t, d = qk.shape
    nc = means_t.shape[2]

    ranksel, count = pl.pallas_call(
        _route_kernel,
        out_shape=(
            jax.ShapeDtypeStruct((b, h, t, nc), jnp.int32),
            jax.ShapeDtypeStruct((b, h, t, 1), jnp.float32),
        ),
        grid=(b, h),
        in_specs=[
            pl.BlockSpec((1, 1, t, d), lambda i, j: (i, j, 0, 0)),
            pl.BlockSpec((1, d, nc), lambda i, j: (j, 0, 0)),
        ],
        out_specs=(
            pl.BlockSpec((1, 1, t, nc), lambda i, j: (i, j, 0, 0)),
            pl.BlockSpec((1, 1, t, 1), lambda i, j: (i, j, 0, 0)),
        ),
        compiler_params=pltpu.CompilerParams(
            dimension_semantics=("parallel", "parallel")),
        interpret=interpret,
    )(qk, means_t)

    ranksel_t = jnp.swapaxes(ranksel, 2, 3).reshape(b, h, nc, t, 1)

    out = pl.pallas_call(
        _attn_kernel,
        out_shape=jax.ShapeDtypeStruct((b, h, t, d), jnp.float32),
        grid=(b, h, nc),
        in_specs=[
            pl.BlockSpec((1, 1, 1, t, 1), lambda i, j, c: (i, j, c, 0, 0)),
            pl.BlockSpec((1, 1, t, d), lambda i, j, c: (i, j, 0, 0)),
            pl.BlockSpec((1, 1, t, d), lambda i, j, c: (i, j, 0, 0)),
            pl.BlockSpec((1, WSZ, d), lambda i, j, c: (j, 0, 0)),
            pl.BlockSpec((1, 1, t, 1), lambda i, j, c: (i, j, 0, 0)),
        ],
        out_specs=pl.BlockSpec((1, 1, t, d), lambda i, j, c: (i, j, 0, 0)),
        scratch_shapes=[pltpu.VMEM((t, d), jnp.float32)],
        compiler_params=pltpu.CompilerParams(
            dimension_semantics=("parallel", "parallel", "arbitrary")),
        interpret=interpret,
    )(ranksel_t, qk, v, rel_t, count)
    return out


@jax.jit
def kernel(qk, v, means, rel_pos_weights):
    means_t = jnp.swapaxes(means, 1, 2)
    rel_t = jnp.swapaxes(rel_pos_weights, 0, 1)
    return _run(qk, v, means_t, rel_t)


# speedup vs baseline: 1.2010x; 1.2010x over previous
import jax
import jax.numpy as jnp
from jax import lax
from jax.experimental import pallas as pl
from jax.experimental.pallas import tpu as pltpu

WSZ = 256
TOKEN_SELF_ATTN_VALUE = -50000.0
MININT = -2147483648
NOT_SELECTED = 1 << 20
CPP = 2


def _excl_cumsum_cols(x, chunk=512):
    t = x.shape[0]
    r_i = lax.broadcasted_iota(jnp.int32, (chunk, chunk), 0)
    c_i = lax.broadcasted_iota(jnp.int32, (chunk, chunk), 1)
    lstrict = (c_i < r_i).astype(jnp.float32)
    carry = jnp.zeros((1, x.shape[1]), jnp.float32)
    outs = []
    for b0 in range(0, t, chunk):
        xb = lax.slice(x, (b0, 0), (b0 + chunk, x.shape[1]))
        outs.append(jnp.dot(lstrict, xb) + carry)
        carry = carry + jnp.sum(xb, axis=0, keepdims=True)
    return jnp.concatenate(outs, axis=0)


def _roll_left(x, s):
    n = x.shape[-1]
    s = s % n
    if s == 0:
        return x
    return jnp.concatenate(
        [lax.slice_in_dim(x, s, n, axis=1), lax.slice_in_dim(x, 0, s, axis=1)],
        axis=1)


def _route_kernel(qk_ref, means_t_ref, ranksel_ref, count_ref):
    t, d = qk_ref.shape[2], qk_ref.shape[3]
    wsz = WSZ
    nc = t // wsz

    qk = qk_ref[0, 0]
    means_t = means_t_ref[0]

    nrm = jnp.sqrt(jnp.sum(qk * qk, axis=1, keepdims=True))
    kn = qk / jnp.maximum(nrm, 1e-12)
    dists = jnp.dot(kn, means_t)

    u = lax.bitcast_convert_type(dists, jnp.int32)
    key = u ^ (lax.shift_right_arithmetic(u, 31) & jnp.int32(0x7FFFFFFF))

    tb = jnp.zeros((1, nc), jnp.int32)
    minint = jnp.int32(MININT)
    for bit in range(31, -1, -1):
        step = minint if bit == 31 else jnp.int32(1 << bit)
        try_b = tb + step
        t_signed = try_b ^ minint
        cnt = jnp.sum((key >= t_signed).astype(jnp.float32), axis=0,
                      keepdims=True)
        tb = jnp.where(cnt >= float(wsz), try_b, tb)
    thr = tb ^ minint

    gt = key > thr
    eq = key == thr
    n_gt = jnp.sum(gt.astype(jnp.float32), axis=0, keepdims=True)
    need = float(wsz) - n_gt
    eq_rank = _excl_cumsum_cols(eq.astype(jnp.float32))
    sel = gt | (eq & (eq_rank < need))
    sel_f = sel.astype(jnp.float32)
    rank = _excl_cumsum_cols(sel_f)

    ranksel_ref[0, 0] = jnp.where(sel, rank.astype(jnp.int32),
                                  jnp.int32(NOT_SELECTED))
    count_ref[0, 0] = jnp.sum(sel_f, axis=1, keepdims=True)


def _window_attn(qk_s, v_s, rel, scale):
    wsz = WSZ
    knrm = jnp.sqrt(jnp.sum(qk_s * qk_s, axis=1, keepdims=True))
    kk = qk_s / jnp.maximum(knrm, 1e-12)
    kr = jnp.concatenate([kk, rel], axis=0)
    de = lax.dot_general(qk_s, kr, (((1,), (1,)), ((), ()))) * scale
    dots = lax.slice(de, (0, 0), (wsz, wsz))
    emb = lax.slice(de, (0, wsz), (wsz, 2 * wsz))

    rows = lax.broadcasted_iota(jnp.int32, (wsz, wsz), 0)
    cols = lax.broadcasted_iota(jnp.int32, (wsz, wsz), 1)
    x = emb
    for kbit in range(8):
        rolled = _roll_left(x, 1 << kbit)
        condb = ((((wsz - 1) - rows) >> kbit) & 1) == 1
        x = jnp.where(condb, rolled, x)
    dots = dots + jnp.where(cols <= rows, x, 0.0)
    dots = jnp.where(rows == cols, TOKEN_SELF_ATTN_VALUE, dots)

    m = jnp.max(dots, axis=1, keepdims=True)
    p = jnp.exp(dots - m)
    sm = p / jnp.sum(p, axis=1, keepdims=True)
    return jnp.dot(sm, v_s)


def _attn_kernel(rankcol_ref, rankrow_ref, qv_ref, rel_t_ref, count_ref,
                 out_ref, acc_ref):
    t, d2 = qv_ref.shape[2], qv_ref.shape[3]
    d = d2 // 2
    wsz = WSZ
    nc = t // wsz
    scale = d ** -0.5
    cc = pl.program_id(2)

    qv = qv_ref[0, 0]
    rel = rel_t_ref[0]
    rank_col = rankcol_ref[0, 0, 0]
    rank_row = rankrow_ref[0, 0, 0]

    w_iota = lax.broadcasted_iota(jnp.int32, (t, wsz), 1)
    w_iota_r = lax.broadcasted_iota(jnp.int32, (wsz, t), 0)

    p_ts = [jnp.where(rank_row[g] == w_iota_r, 1.0, 0.0) for g in range(CPP)]
    p_t2 = jnp.concatenate(p_ts, axis=0)
    qv_s2 = jnp.dot(p_t2, qv)

    bos = []
    for g in range(CPP):
        qk_s = lax.slice(qv_s2, (g * wsz, 0), ((g + 1) * wsz, d))
        v_s = lax.slice(qv_s2, (g * wsz, d), ((g + 1) * wsz, d2))
        bos.append(_window_attn(qk_s, v_s, rel, scale))
    bo2 = jnp.concatenate(bos, axis=0)

    p_ms = [jnp.where(rank_col[g] == w_iota, 1.0, 0.0) for g in range(CPP)]
    p_m2 = jnp.concatenate(p_ms, axis=1)
    contrib = jnp.dot(p_m2, bo2)

    @pl.when(cc == 0)
    def _():
        acc_ref[...] = contrib

    @pl.when(cc > 0)
    def _():
        acc_ref[...] = acc_ref[...] + contrib

    @pl.when(cc == nc // CPP - 1)
    def _():
        out_ref[0, 0] = acc_ref[...] / (count_ref[0, 0] + 1e-05)


def _run(qk, v, means_t, rel_t, interpret=False):
    b, h, t, d = qk.shape
    nc = means_t.shape[2]

    ranksel, count = pl.pallas_call(
        _route_kernel,
        out_shape=(
            jax.ShapeDtypeStruct((b, h, t, nc), jnp.int32),
            jax.ShapeDtypeStruct((b, h, t, 1), jnp.float32),
        ),
        grid=(b, h),
        in_specs=[
            pl.BlockSpec((1, 1, t, d), lambda i, j: (i, j, 0, 0)),
            pl.BlockSpec((1, d, nc), lambda i, j: (j, 0, 0)),
        ],
        out_specs=(
            pl.BlockSpec((1, 1, t, nc), lambda i, j: (i, j, 0, 0)),
            pl.BlockSpec((1, 1, t, 1), lambda i, j: (i, j, 0, 0)),
        ),
        compiler_params=pltpu.CompilerParams(
            dimension_semantics=("parallel", "parallel")),
        interpret=interpret,
    )(qk, means_t)

    ranksel_t = jnp.swapaxes(ranksel, 2, 3)
    rank_col = ranksel_t.reshape(b, h, nc // CPP, CPP, t, 1)
    rank_row = ranksel_t.reshape(b, h, nc // CPP, CPP, 1, t)
    qv = jnp.concatenate([qk, v], axis=3)

    out = pl.pallas_call(
        _attn_kernel,
        out_shape=jax.ShapeDtypeStruct((b, h, t, d), jnp.float32),
        grid=(b, h, nc // CPP),
        in_specs=[
            pl.BlockSpec((1, 1, 1, CPP, t, 1),
                         lambda i, j, c: (i, j, c, 0, 0, 0)),
            pl.BlockSpec((1, 1, 1, CPP, 1, t),
                         lambda i, j, c: (i, j, c, 0, 0, 0)),
            pl.BlockSpec((1, 1, t, 2 * d), lambda i, j, c: (i, j, 0, 0)),
            pl.BlockSpec((1, WSZ, d), lambda i, j, c: (j, 0, 0)),
            pl.BlockSpec((1, 1, t, 1), lambda i, j, c: (i, j, 0, 0)),
        ],
        out_specs=pl.BlockSpec((1, 1, t, d), lambda i, j, c: (i, j, 0, 0)),
        scratch_shapes=[pltpu.VMEM((t, d), jnp.float32)],
        compiler_params=pltpu.CompilerParams(
            dimension_semantics=("parallel", "parallel", "arbitrary")),
        interpret=interpret,
    )(rank_col, rank_row, qv, rel_t, count)
    return out


@jax.jit
def kernel(qk, v, means, rel_pos_weights):
    means_t = jnp.swapaxes(means, 1, 2)
    rel_t = jnp.swapaxes(rel_pos_weights, 0, 1)
    return _run(qk, v, means_t, rel_t)


# speedup vs baseline: 1.2274x; 1.0220x over previous
import jax
import jax.numpy as jnp
from jax import lax
from jax.experimental import pallas as pl
from jax.experimental.pallas import tpu as pltpu

WSZ = 256
TOKEN_SELF_ATTN_VALUE = -50000.0
MININT = -2147483648
NOT_SELECTED = 1 << 20
CPP = 4


def _excl_cumsum_cols(x, chunk=512):
    t = x.shape[0]
    r_i = lax.broadcasted_iota(jnp.int32, (chunk, chunk), 0)
    c_i = lax.broadcasted_iota(jnp.int32, (chunk, chunk), 1)
    lstrict = (c_i < r_i).astype(jnp.float32)
    carry = jnp.zeros((1, x.shape[1]), jnp.float32)
    outs = []
    for b0 in range(0, t, chunk):
        xb = lax.slice(x, (b0, 0), (b0 + chunk, x.shape[1]))
        outs.append(jnp.dot(lstrict, xb) + carry)
        carry = carry + jnp.sum(xb, axis=0, keepdims=True)
    return jnp.concatenate(outs, axis=0)


def _roll_left(x, s):
    n = x.shape[-1]
    s = s % n
    if s == 0:
        return x
    return jnp.concatenate(
        [lax.slice_in_dim(x, s, n, axis=1), lax.slice_in_dim(x, 0, s, axis=1)],
        axis=1)


def _route_kernel(qk_ref, means_t_ref, ranksel_ref, count_ref):
    t, d = qk_ref.shape[2], qk_ref.shape[3]
    wsz = WSZ
    nc = t // wsz

    qk = qk_ref[0, 0]
    means_t = means_t_ref[0]

    nrm = jnp.sqrt(jnp.sum(qk * qk, axis=1, keepdims=True))
    kn = qk / jnp.maximum(nrm, 1e-12)
    dists = jnp.dot(kn, means_t)

    u = lax.bitcast_convert_type(dists, jnp.int32)
    key = u ^ (lax.shift_right_arithmetic(u, 31) & jnp.int32(0x7FFFFFFF))

    tb = jnp.zeros((1, nc), jnp.int32)
    minint = jnp.int32(MININT)
    for bit in range(31, -1, -1):
        step = minint if bit == 31 else jnp.int32(1 << bit)
        try_b = tb + step
        t_signed = try_b ^ minint
        cnt = jnp.sum((key >= t_signed).astype(jnp.float32), axis=0,
                      keepdims=True)
        tb = jnp.where(cnt >= float(wsz), try_b, tb)
    thr = tb ^ minint

    gt = key > thr
    eq = key == thr
    n_gt = jnp.sum(gt.astype(jnp.float32), axis=0, keepdims=True)
    need = float(wsz) - n_gt
    eq_rank = _excl_cumsum_cols(eq.astype(jnp.float32))
    sel = gt | (eq & (eq_rank < need))
    sel_f = sel.astype(jnp.float32)
    rank = _excl_cumsum_cols(sel_f)

    ranksel_ref[0, 0] = jnp.where(sel, rank.astype(jnp.int32),
                                  jnp.int32(NOT_SELECTED))
    count_ref[0, 0] = jnp.sum(sel_f, axis=1, keepdims=True)


def _window_attn(qk_s, v_s, rel, scale):
    wsz = WSZ
    knrm = jnp.sqrt(jnp.sum(qk_s * qk_s, axis=1, keepdims=True))
    kk = qk_s / jnp.maximum(knrm, 1e-12)
    kr = jnp.concatenate([kk, rel], axis=0)
    de = lax.dot_general(qk_s, kr, (((1,), (1,)), ((), ()))) * scale
    dots = lax.slice(de, (0, 0), (wsz, wsz))
    emb = lax.slice(de, (0, wsz), (wsz, 2 * wsz))

    rows = lax.broadcasted_iota(jnp.int32, (wsz, wsz), 0)
    cols = lax.broadcasted_iota(jnp.int32, (wsz, wsz), 1)
    x = emb
    for kbit in range(8):
        rolled = _roll_left(x, 1 << kbit)
        condb = ((((wsz - 1) - rows) >> kbit) & 1) == 1
        x = jnp.where(condb, rolled, x)
    dots = dots + jnp.where(cols <= rows, x, 0.0)
    dots = jnp.where(rows == cols, TOKEN_SELF_ATTN_VALUE, dots)

    m = jnp.max(dots, axis=1, keepdims=True)
    p = jnp.exp(dots - m)
    sm = p / jnp.sum(p, axis=1, keepdims=True)
    return jnp.dot(sm, v_s)


def _attn_kernel(rankcol_ref, rankrow_ref, qv_ref, rel_t_ref, count_ref,
                 out_ref, acc_ref):
    t, d2 = qv_ref.shape[2], qv_ref.shape[3]
    d = d2 // 2
    wsz = WSZ
    nc = t // wsz
    scale = d ** -0.5
    cc = pl.program_id(2)

    qv = qv_ref[0, 0]
    rel = rel_t_ref[0]
    rank_col = rankcol_ref[0, 0, 0]
    rank_row = rankrow_ref[0, 0, 0]

    w_iota = lax.broadcasted_iota(jnp.int32, (t, wsz), 1)
    w_iota_r = lax.broadcasted_iota(jnp.int32, (wsz, t), 0)

    p_ts = [jnp.where(rank_row[g] == w_iota_r, 1.0, 0.0) for g in range(CPP)]
    p_t2 = jnp.concatenate(p_ts, axis=0)
    qv_s2 = jnp.dot(p_t2, qv)

    bos = []
    for g in range(CPP):
        qk_s = lax.slice(qv_s2, (g * wsz, 0), ((g + 1) * wsz, d))
        v_s = lax.slice(qv_s2, (g * wsz, d), ((g + 1) * wsz, d2))
        bos.append(_window_attn(qk_s, v_s, rel, scale))
    bo2 = jnp.concatenate(bos, axis=0)

    p_ms = [jnp.where(rank_col[g] == w_iota, 1.0, 0.0) for g in range(CPP)]
    p_m2 = jnp.concatenate(p_ms, axis=1)
    contrib = jnp.dot(p_m2, bo2)

    @pl.when(cc == 0)
    def _():
        acc_ref[...] = contrib

    @pl.when(cc > 0)
    def _():
        acc_ref[...] = acc_ref[...] + contrib

    @pl.when(cc == nc // CPP - 1)
    def _():
        out_ref[0, 0] = acc_ref[...] / (count_ref[0, 0] + 1e-05)


def _run(qk, v, means_t, rel_t, interpret=False):
    b, h, t, d = qk.shape
    nc = means_t.shape[2]

    ranksel, count = pl.pallas_call(
        _route_kernel,
        out_shape=(
            jax.ShapeDtypeStruct((b, h, t, nc), jnp.int32),
            jax.ShapeDtypeStruct((b, h, t, 1), jnp.float32),
        ),
        grid=(b, h),
        in_specs=[
            pl.BlockSpec((1, 1, t, d), lambda i, j: (i, j, 0, 0)),
            pl.BlockSpec((1, d, nc), lambda i, j: (j, 0, 0)),
        ],
        out_specs=(
            pl.BlockSpec((1, 1, t, nc), lambda i, j: (i, j, 0, 0)),
            pl.BlockSpec((1, 1, t, 1), lambda i, j: (i, j, 0, 0)),
        ),
        compiler_params=pltpu.CompilerParams(
            dimension_semantics=("parallel", "parallel")),
        interpret=interpret,
    )(qk, means_t)

    ranksel_t = jnp.swapaxes(ranksel, 2, 3)
    rank_col = ranksel_t.reshape(b, h, nc // CPP, CPP, t, 1)
    rank_row = ranksel_t.reshape(b, h, nc // CPP, CPP, 1, t)
    qv = jnp.concatenate([qk, v], axis=3)

    out = pl.pallas_call(
        _attn_kernel,
        out_shape=jax.ShapeDtypeStruct((b, h, t, d), jnp.float32),
        grid=(b, h, nc // CPP),
        in_specs=[
            pl.BlockSpec((1, 1, 1, CPP, t, 1),
                         lambda i, j, c: (i, j, c, 0, 0, 0)),
            pl.BlockSpec((1, 1, 1, CPP, 1, t),
                         lambda i, j, c: (i, j, c, 0, 0, 0)),
            pl.BlockSpec((1, 1, t, 2 * d), lambda i, j, c: (i, j, 0, 0)),
            pl.BlockSpec((1, WSZ, d), lambda i, j, c: (j, 0, 0)),
            pl.BlockSpec((1, 1, t, 1), lambda i, j, c: (i, j, 0, 0)),
        ],
        out_specs=pl.BlockSpec((1, 1, t, d), lambda i, j, c: (i, j, 0, 0)),
        scratch_shapes=[pltpu.VMEM((t, d), jnp.float32)],
        compiler_params=pltpu.CompilerParams(
            dimension_semantics=("parallel", "parallel", "arbitrary")),
        interpret=interpret,
    )(rank_col, rank_row, qv, rel_t, count)
    return out


@jax.jit
def kernel(qk, v, means, rel_pos_weights):
    means_t = jnp.swapaxes(means, 1, 2)
    rel_t = jnp.swapaxes(rel_pos_weights, 0, 1)
    return _run(qk, v, means_t, rel_t)


# speedup vs baseline: 1.2434x; 1.0130x over previous
import jax
import jax.numpy as jnp
from jax import lax
from jax.experimental import pallas as pl
from jax.experimental.pallas import tpu as pltpu

WSZ = 256
TOKEN_SELF_ATTN_VALUE = -50000.0
MININT = -2147483648
NOT_SELECTED = 1 << 20
CPP = 4


def _excl_cumsum_cols(x, chunk=512):
    t = x.shape[0]
    r_i = lax.broadcasted_iota(jnp.int32, (chunk, chunk), 0)
    c_i = lax.broadcasted_iota(jnp.int32, (chunk, chunk), 1)
    lstrict = (c_i < r_i).astype(jnp.float32)
    carry = jnp.zeros((1, x.shape[1]), jnp.float32)
    outs = []
    for b0 in range(0, t, chunk):
        xb = lax.slice(x, (b0, 0), (b0 + chunk, x.shape[1]))
        outs.append(jnp.dot(lstrict, xb) + carry)
        carry = carry + jnp.sum(xb, axis=0, keepdims=True)
    return jnp.concatenate(outs, axis=0)


def _roll_left(x, s):
    n = x.shape[-1]
    s = s % n
    if s == 0:
        return x
    return jnp.concatenate(
        [lax.slice_in_dim(x, s, n, axis=1), lax.slice_in_dim(x, 0, s, axis=1)],
        axis=1)


def _route_kernel(qk_ref, v_ref, means_t_ref, ranksel_ref, count_ref,
                  qv_ref):
    t, d = qk_ref.shape[2], qk_ref.shape[3]
    wsz = WSZ
    nc = t // wsz

    qk = qk_ref[0, 0]
    means_t = means_t_ref[0]
    qv_ref[0, 0] = jnp.concatenate([qk, v_ref[0, 0]], axis=1)

    nrm = jnp.sqrt(jnp.sum(qk * qk, axis=1, keepdims=True))
    kn = qk / jnp.maximum(nrm, 1e-12)
    dists = jnp.dot(kn, means_t)

    u = lax.bitcast_convert_type(dists, jnp.int32)
    key = u ^ (lax.shift_right_arithmetic(u, 31) & jnp.int32(0x7FFFFFFF))

    tb = jnp.zeros((1, nc), jnp.int32)
    minint = jnp.int32(MININT)
    for bit in range(31, -1, -1):
        step = minint if bit == 31 else jnp.int32(1 << bit)
        try_b = tb + step
        t_signed = try_b ^ minint
        cnt = jnp.sum((key >= t_signed).astype(jnp.float32), axis=0,
                      keepdims=True)
        tb = jnp.where(cnt >= float(wsz), try_b, tb)
    thr = tb ^ minint

    gt = key > thr
    eq = key == thr
    n_gt = jnp.sum(gt.astype(jnp.float32), axis=0, keepdims=True)
    need = float(wsz) - n_gt
    eq_rank = _excl_cumsum_cols(eq.astype(jnp.float32))
    sel = gt | (eq & (eq_rank < need))
    sel_f = sel.astype(jnp.float32)
    rank = _excl_cumsum_cols(sel_f)

    ranksel = jnp.where(sel, rank.astype(jnp.int32), jnp.int32(NOT_SELECTED))
    ranksel_ref[0, 0] = jnp.transpose(ranksel)
    count_ref[0, 0] = jnp.sum(sel_f, axis=1, keepdims=True)


def _window_attn(qk_s, v_s, rel, scale):
    wsz = WSZ
    knrm = jnp.sqrt(jnp.sum(qk_s * qk_s, axis=1, keepdims=True))
    kk = qk_s / jnp.maximum(knrm, 1e-12)
    kr = jnp.concatenate([kk, rel], axis=0)
    de = lax.dot_general(qk_s, kr, (((1,), (1,)), ((), ()))) * scale
    dots = lax.slice(de, (0, 0), (wsz, wsz))
    emb = lax.slice(de, (0, wsz), (wsz, 2 * wsz))

    rows = lax.broadcasted_iota(jnp.int32, (wsz, wsz), 0)
    cols = lax.broadcasted_iota(jnp.int32, (wsz, wsz), 1)
    x = emb
    for kbit in range(8):
        rolled = _roll_left(x, 1 << kbit)
        condb = ((((wsz - 1) - rows) >> kbit) & 1) == 1
        x = jnp.where(condb, rolled, x)
    dots = dots + jnp.where(cols <= rows, x, 0.0)
    dots = jnp.where(rows == cols, TOKEN_SELF_ATTN_VALUE, dots)

    m = jnp.max(dots, axis=1, keepdims=True)
    p = jnp.exp(dots - m)
    sm = p / jnp.sum(p, axis=1, keepdims=True)
    return jnp.dot(sm, v_s)


def _attn_kernel(rankcol_ref, rankrow_ref, qv_ref, rel_t_ref, count_ref,
                 out_ref, acc_ref):
    t, d2 = qv_ref.shape[2], qv_ref.shape[3]
    d = d2 // 2
    wsz = WSZ
    nc = t // wsz
    scale = d ** -0.5
    cc = pl.program_id(2)

    qv = qv_ref[0, 0]
    rel = rel_t_ref[0]
    rank_col = rankcol_ref[0, 0, 0]
    rank_row = rankrow_ref[0, 0, 0]

    w_iota = lax.broadcasted_iota(jnp.int32, (t, wsz), 1)
    w_iota_r = lax.broadcasted_iota(jnp.int32, (wsz, t), 0)

    p_ts = [jnp.where(rank_row[g] == w_iota_r, 1.0, 0.0) for g in range(CPP)]
    p_t2 = jnp.concatenate(p_ts, axis=0)
    qv_s2 = jnp.dot(p_t2, qv)

    bos = []
    for g in range(CPP):
        qk_s = lax.slice(qv_s2, (g * wsz, 0), ((g + 1) * wsz, d))
        v_s = lax.slice(qv_s2, (g * wsz, d), ((g + 1) * wsz, d2))
        bos.append(_window_attn(qk_s, v_s, rel, scale))
    bo2 = jnp.concatenate(bos, axis=0)

    p_ms = [jnp.where(rank_col[g] == w_iota, 1.0, 0.0) for g in range(CPP)]
    p_m2 = jnp.concatenate(p_ms, axis=1)
    contrib = jnp.dot(p_m2, bo2)

    @pl.when(cc == 0)
    def _():
        acc_ref[...] = contrib

    @pl.when(cc > 0)
    def _():
        acc_ref[...] = acc_ref[...] + contrib

    @pl.when(cc == nc // CPP - 1)
    def _():
        out_ref[0, 0] = acc_ref[...] / (count_ref[0, 0] + 1e-05)


def _run(qk, v, means_t, rel_t, interpret=False):
    b, h, t, d = qk.shape
    nc = means_t.shape[2]

    ranksel_t, count, qv = pl.pallas_call(
        _route_kernel,
        out_shape=(
            jax.ShapeDtypeStruct((b, h, nc, t), jnp.int32),
            jax.ShapeDtypeStruct((b, h, t, 1), jnp.float32),
            jax.ShapeDtypeStruct((b, h, t, 2 * d), jnp.float32),
        ),
        grid=(b, h),
        in_specs=[
            pl.BlockSpec((1, 1, t, d), lambda i, j: (i, j, 0, 0)),
            pl.BlockSpec((1, 1, t, d), lambda i, j: (i, j, 0, 0)),
            pl.BlockSpec((1, d, nc), lambda i, j: (j, 0, 0)),
        ],
        out_specs=(
            pl.BlockSpec((1, 1, nc, t), lambda i, j: (i, j, 0, 0)),
            pl.BlockSpec((1, 1, t, 1), lambda i, j: (i, j, 0, 0)),
            pl.BlockSpec((1, 1, t, 2 * d), lambda i, j: (i, j, 0, 0)),
        ),
        compiler_params=pltpu.CompilerParams(
            dimension_semantics=("parallel", "parallel")),
        interpret=interpret,
    )(qk, v, means_t)

    rank_col = ranksel_t.reshape(b, h, nc // CPP, CPP, t, 1)
    rank_row = ranksel_t.reshape(b, h, nc // CPP, CPP, 1, t)

    out = pl.pallas_call(
        _attn_kernel,
        out_shape=jax.ShapeDtypeStruct((b, h, t, d), jnp.float32),
        grid=(b, h, nc // CPP),
        in_specs=[
            pl.BlockSpec((1, 1, 1, CPP, t, 1),
                         lambda i, j, c: (i, j, c, 0, 0, 0)),
            pl.BlockSpec((1, 1, 1, CPP, 1, t),
                         lambda i, j, c: (i, j, c, 0, 0, 0)),
            pl.BlockSpec((1, 1, t, 2 * d), lambda i, j, c: (i, j, 0, 0)),
            pl.BlockSpec((1, WSZ, d), lambda i, j, c: (j, 0, 0)),
            pl.BlockSpec((1, 1, t, 1), lambda i, j, c: (i, j, 0, 0)),
        ],
        out_specs=pl.BlockSpec((1, 1, t, d), lambda i, j, c: (i, j, 0, 0)),
        scratch_shapes=[pltpu.VMEM((t, d), jnp.float32)],
        compiler_params=pltpu.CompilerParams(
            dimension_semantics=("parallel", "parallel", "arbitrary")),
        interpret=interpret,
    )(rank_col, rank_row, qv, rel_t, count)
    return out


@jax.jit
def kernel(qk, v, means, rel_pos_weights):
    means_t = jnp.swapaxes(means, 1, 2)
    rel_t = jnp.swapaxes(rel_pos_weights, 0, 1)
    return _run(qk, v, means_t, rel_t)


# speedup vs baseline: 1.4780x; 1.1887x over previous
import jax
import jax.numpy as jnp
from jax import lax
from jax.experimental import pallas as pl
from jax.experimental.pallas import tpu as pltpu

WSZ = 256
TOKEN_SELF_ATTN_VALUE = -50000.0
MININT = -2147483648
NOT_SELECTED = 1 << 20
CPP = 4


def _shiftr_lanes(x, s):
    n = x.shape[-1]
    z = jnp.zeros(x.shape[:-1] + (s,), x.dtype)
    return jnp.concatenate([z, lax.slice_in_dim(x, 0, n - s, axis=-1)], axis=-1)


def _excl_cumsum_lanes(x):
    n = x.shape[-1]
    y = x
    s = 1
    while s < n:
        y = y + _shiftr_lanes(y, s)
        s *= 2
    return y - x


def _roll_left(x, s):
    n = x.shape[-1]
    s = s % n
    if s == 0:
        return x
    return jnp.concatenate(
        [lax.slice_in_dim(x, s, n, axis=1), lax.slice_in_dim(x, 0, s, axis=1)],
        axis=1)


def _route_kernel(qk_ref, v_ref, means_t_ref, ranksel_ref, count_ref,
                  qv_ref):
    t, d = qk_ref.shape[2], qk_ref.shape[3]
    wsz = WSZ
    nc = t // wsz

    qk = qk_ref[0, 0]
    means_t = means_t_ref[0]
    qv_ref[0, 0] = jnp.concatenate([qk, v_ref[0, 0]], axis=1)

    nrm = jnp.sqrt(jnp.sum(qk * qk, axis=1, keepdims=True))
    kn = qk / jnp.maximum(nrm, 1e-12)
    dists = jnp.dot(kn, means_t)
    dists_t = jnp.transpose(dists)

    u = lax.bitcast_convert_type(dists_t, jnp.int32)
    key = u ^ (lax.shift_right_arithmetic(u, 31) & jnp.int32(0x7FFFFFFF))

    tb = jnp.zeros((nc, 1), jnp.int32)
    minint = jnp.int32(MININT)
    for bit in range(31, -1, -1):
        step = minint if bit == 31 else jnp.int32(1 << bit)
        try_b = tb + step
        t_signed = try_b ^ minint
        cnt = jnp.sum((key >= t_signed).astype(jnp.float32), axis=1,
                      keepdims=True)
        tb = jnp.where(cnt >= float(wsz), try_b, tb)
    thr = tb ^ minint

    gt = key > thr
    eq = key == thr
    n_gt = jnp.sum(gt.astype(jnp.float32), axis=1, keepdims=True)
    need = float(wsz) - n_gt
    eq_rank = _excl_cumsum_lanes(eq.astype(jnp.float32))
    sel = gt | (eq & (eq_rank < need))
    sel_f = sel.astype(jnp.float32)
    rank = _excl_cumsum_lanes(sel_f)

    ranksel_ref[0, 0] = jnp.where(sel, rank.astype(jnp.int32),
                                  jnp.int32(NOT_SELECTED))
    count_ref[0, 0] = jnp.transpose(
        jnp.sum(sel_f, axis=0, keepdims=True))


def _window_attn(qk_s, v_s, rel, scale):
    wsz = WSZ
    knrm = jnp.sqrt(jnp.sum(qk_s * qk_s, axis=1, keepdims=True))
    kk = qk_s / jnp.maximum(knrm, 1e-12)
    kr = jnp.concatenate([kk, rel], axis=0)
    de = lax.dot_general(qk_s, kr, (((1,), (1,)), ((), ()))) * scale
    dots = lax.slice(de, (0, 0), (wsz, wsz))
    emb = lax.slice(de, (0, wsz), (wsz, 2 * wsz))

    rows = lax.broadcasted_iota(jnp.int32, (wsz, wsz), 0)
    cols = lax.broadcasted_iota(jnp.int32, (wsz, wsz), 1)
    x = emb
    for kbit in range(8):
        rolled = _roll_left(x, 1 << kbit)
        condb = ((((wsz - 1) - rows) >> kbit) & 1) == 1
        x = jnp.where(condb, rolled, x)
    dots = dots + jnp.where(cols <= rows, x, 0.0)
    dots = jnp.where(rows == cols, TOKEN_SELF_ATTN_VALUE, dots)

    m = jnp.max(dots, axis=1, keepdims=True)
    p = jnp.exp(dots - m)
    sm = p / jnp.sum(p, axis=1, keepdims=True)
    return jnp.dot(sm, v_s)


def _attn_kernel(rankcol_ref, rankrow_ref, qv_ref, rel_t_ref, count_ref,
                 out_ref, acc_ref):
    t, d2 = qv_ref.shape[2], qv_ref.shape[3]
    d = d2 // 2
    wsz = WSZ
    nc = t // wsz
    scale = d ** -0.5
    cc = pl.program_id(2)

    qv = qv_ref[0, 0]
    rel = rel_t_ref[0]
    rank_col = rankcol_ref[0, 0, 0]
    rank_row = rankrow_ref[0, 0, 0]

    w_iota = lax.broadcasted_iota(jnp.int32, (t, wsz), 1)
    w_iota_r = lax.broadcasted_iota(jnp.int32, (wsz, t), 0)

    p_ts = [jnp.where(rank_row[g] == w_iota_r, 1.0, 0.0) for g in range(CPP)]
    p_t2 = jnp.concatenate(p_ts, axis=0)
    qv_s2 = jnp.dot(p_t2, qv)

    bos = []
    for g in range(CPP):
        qk_s = lax.slice(qv_s2, (g * wsz, 0), ((g + 1) * wsz, d))
        v_s = lax.slice(qv_s2, (g * wsz, d), ((g + 1) * wsz, d2))
        bos.append(_window_attn(qk_s, v_s, rel, scale))
    bo2 = jnp.concatenate(bos, axis=0)

    p_ms = [jnp.where(rank_col[g] == w_iota, 1.0, 0.0) for g in range(CPP)]
    p_m2 = jnp.concatenate(p_ms, axis=1)
    contrib = jnp.dot(p_m2, bo2)

    @pl.when(cc == 0)
    def _():
        acc_ref[...] = contrib

    @pl.when(cc > 0)
    def _():
        acc_ref[...] = acc_ref[...] + contrib

    @pl.when(cc == nc // CPP - 1)
    def _():
        out_ref[0, 0] = acc_ref[...] / (count_ref[0, 0] + 1e-05)


def _run(qk, v, means_t, rel_t, interpret=False):
    b, h, t, d = qk.shape
    nc = means_t.shape[2]

    ranksel_t, count, qv = pl.pallas_call(
        _route_kernel,
        out_shape=(
            jax.ShapeDtypeStruct((b, h, nc, t), jnp.int32),
            jax.ShapeDtypeStruct((b, h, t, 1), jnp.float32),
            jax.ShapeDtypeStruct((b, h, t, 2 * d), jnp.float32),
        ),
        grid=(b, h),
        in_specs=[
            pl.BlockSpec((1, 1, t, d), lambda i, j: (i, j, 0, 0)),
            pl.BlockSpec((1, 1, t, d), lambda i, j: (i, j, 0, 0)),
            pl.BlockSpec((1, d, nc), lambda i, j: (j, 0, 0)),
        ],
        out_specs=(
            pl.BlockSpec((1, 1, nc, t), lambda i, j: (i, j, 0, 0)),
            pl.BlockSpec((1, 1, t, 1), lambda i, j: (i, j, 0, 0)),
            pl.BlockSpec((1, 1, t, 2 * d), lambda i, j: (i, j, 0, 0)),
        ),
        compiler_params=pltpu.CompilerParams(
            dimension_semantics=("parallel", "parallel")),
        interpret=interpret,
    )(qk, v, means_t)

    rank_col = ranksel_t.reshape(b, h, nc // CPP, CPP, t, 1)
    rank_row = ranksel_t.reshape(b, h, nc // CPP, CPP, 1, t)

    out = pl.pallas_call(
        _attn_kernel,
        out_shape=jax.ShapeDtypeStruct((b, h, t, d), jnp.float32),
        grid=(b, h, nc // CPP),
        in_specs=[
            pl.BlockSpec((1, 1, 1, CPP, t, 1),
                         lambda i, j, c: (i, j, c, 0, 0, 0)),
            pl.BlockSpec((1, 1, 1, CPP, 1, t),
                         lambda i, j, c: (i, j, c, 0, 0, 0)),
            pl.BlockSpec((1, 1, t, 2 * d), lambda i, j, c: (i, j, 0, 0)),
            pl.BlockSpec((1, WSZ, d), lambda i, j, c: (j, 0, 0)),
            pl.BlockSpec((1, 1, t, 1), lambda i, j, c: (i, j, 0, 0)),
        ],
        out_specs=pl.BlockSpec((1, 1, t, d), lambda i, j, c: (i, j, 0, 0)),
        scratch_shapes=[pltpu.VMEM((t, d), jnp.float32)],
        compiler_params=pltpu.CompilerParams(
            dimension_semantics=("parallel", "parallel", "arbitrary")),
        interpret=interpret,
    )(rank_col, rank_row, qv, rel_t, count)
    return out


@jax.jit
def kernel(qk, v, means, rel_pos_weights):
    means_t = jnp.swapaxes(means, 1, 2)
    rel_t = jnp.swapaxes(rel_pos_weights, 0, 1)
    return _run(qk, v, means_t, rel_t)
